# baseline (speedup 1.0000x reference)
import functools

import jax
import jax.numpy as jnp
from jax import lax
from jax.experimental import pallas as pl
from jax.experimental.pallas import tpu as pltpu
from jax.experimental.pallas import tpu_sc as plsc


_NW = 32


def _pick_chunk(b, d):
    best = None
    for div in (1, 2, 4, 5, 8, 10, 16, 20, 25, 40, 50):
        if b % div:
            continue
        c = b // div
        if c % 8 == 0 and c * (d + 1) * 4 <= 380_000:
            best = c
            break
    if best is None:
        raise ValueError(f"no chunk for b={b} d={d}")
    return best


def _sc_gather(table, idx):
    m = idx.shape[0]
    b = m // _NW
    two_d = table.ndim == 2
    d = table.shape[1] if two_d else 1
    c = _pick_chunk(b, d)
    n_chunks = b // c
    row_shape = (c, d) if two_d else (c,)
    mesh = plsc.VectorSubcoreMesh(core_axis_name="c", subcore_axis_name="s")

    @functools.partial(
        pl.kernel, mesh=mesh,
        out_type=jax.ShapeDtypeStruct(idx.shape[:1] + table.shape[1:],
                                      table.dtype),
        scratch_types=[
            pltpu.VMEM((c,), jnp.int32),
            pltpu.VMEM(row_shape, table.dtype),
            pltpu.SemaphoreType.DMA,
        ],
    )
    def k(table_hbm, idx_hbm, out_hbm, idx_v, rows_v, sem):
        wid = lax.axis_index("s") * 2 + lax.axis_index("c")
        base = wid * b
        for j in range(n_chunks):
            o = base + j * c
            pltpu.sync_copy(idx_hbm.at[pl.ds(o, c)], idx_v)
            pltpu.async_copy(table_hbm.at[idx_v], rows_v, sem).wait()
            pltpu.sync_copy(rows_v, out_hbm.at[pl.ds(o, c)])

    return k(table, idx)


def _gather_pair(table, ia, ib):
    out = _sc_gather(table, jnp.concatenate([ia, ib]))
    return out[: ia.shape[0]], out[ia.shape[0]:]


def _edge_geom(pos, deg, ia, ib, n):
    e = ia.shape[0]
    table = jnp.concatenate([pos[:, 0], pos[:, 1], deg.astype(jnp.float32)])
    idx = jnp.concatenate([ia, ib, n + ia, n + ib, 2 * n + ia, 2 * n + ib])
    out = _sc_gather(table, idx)
    px_s, px_d, py_s, py_d, dg_s, dg_d = [out[i * e:(i + 1) * e]
                                          for i in range(6)]
    return px_s, py_s, px_d, py_d, dg_s, dg_d


def _perm16(v, perm):
    dnums = lax.GatherDimensionNumbers(offset_dims=(),
                                       collapsed_slice_dims=(0,),
                                       start_index_map=(0,))
    return lax.gather(v, perm[:, None], dnums, slice_sizes=(1,),
                      mode=lax.GatherScatterMode.PROMISE_IN_BOUNDS)


def _lexmax_scatter(row, col, ww, n):
    m = row.shape[0]
    b = m // _NW
    npad = ((n + 256) // 256) * 256
    mesh = plsc.VectorSubcoreMesh(core_axis_name="c", subcore_axis_name="s")
    initw = jnp.full((npad,), -jnp.inf, jnp.float32)
    initc = jnp.full((npad,), -1, jnp.int32)

    @functools.partial(
        pl.kernel, mesh=mesh,
        compiler_params=pltpu.CompilerParams(needs_layout_passes=False),
        out_type=(jax.ShapeDtypeStruct((_NW, npad), jnp.float32),
                  jax.ShapeDtypeStruct((_NW, npad), jnp.int32)),
        scratch_types=[
            pltpu.VMEM((b,), jnp.int32),
            pltpu.VMEM((b,), jnp.int32),
            pltpu.VMEM((b,), jnp.float32),
            pltpu.VMEM((npad,), jnp.float32),
            pltpu.VMEM((npad,), jnp.int32),
            pltpu.SemaphoreType.DMA,
        ],
    )
    def k(row_hbm, col_hbm, ww_hbm, iw_hbm, ic_hbm, outw_hbm, outc_hbm,
          idx_v, col_v, ww_v, accw_v, accc_v, sem):
        wid = lax.axis_index("s") * 2 + lax.axis_index("c")
        base = wid * b
        pltpu.sync_copy(iw_hbm, accw_v)
        pltpu.sync_copy(ic_hbm, accc_v)
        pltpu.sync_copy(row_hbm.at[pl.ds(base, b)], idx_v)
        pltpu.sync_copy(col_hbm.at[pl.ds(base, b)], col_v)
        pltpu.sync_copy(ww_hbm.at[pl.ds(base, b)], ww_v)
        lane = lax.iota(jnp.int32, 16)

        def it(i, carry):
            s = pl.ds(i * 16, 16)
            key = idx_v[s] * 16384 + col_v[s]
            ks, ws = plsc.sort_key_val(key, ww_v[s])
            seg = ks >> 14
            cs = ks & 16383
            for d in (1, 2, 4, 8):
                perm = jnp.maximum(lane - d, 0)
                pi = _perm16(seg, perm)
                pw = _perm16(ws, perm)
                pc = _perm16(cs, perm)
                same = pi == seg
                win = same & ((pw > ws) | ((pw == ws) & (pc > cs)))
                ws = jnp.where(win, pw, ws)
                cs = jnp.where(win, pc, cs)
            nxt = _perm16(seg, jnp.minimum(lane + 1, 15))
            last = (seg != nxt) | (lane == 15)
            cw = plsc.load_gather(accw_v, [seg])
            cc = plsc.load_gather(accc_v, [seg])
            win2 = (ws > cw) | ((ws == cw) & (cs > cc))
            msk = last & win2
            plsc.store_scatter(accw_v, [seg], ws, mask=msk)
            plsc.store_scatter(accc_v, [seg], cs, mask=msk)
            return carry

        lax.fori_loop(0, b // 16, it, 0)
        pltpu.sync_copy(accw_v, outw_hbm.at[wid])
        pltpu.sync_copy(accc_v, outc_hbm.at[wid])

    outw, outc = k(row, col, ww, initw, initc)
    mw = jnp.max(outw[:, :n], axis=0)
    pr = jnp.max(jnp.where(outw[:, :n] == mw[None, :], outc[:, :n], -1),
                 axis=0)
    return mw, pr


def _segmax_rows(msg4, idx, n, out_c):
    e = idx.shape[0]
    g_cnt = out_c // 8
    r_cnt = _NW // g_cnt
    b = e // r_cnt
    chunk = 2500
    n_chunks = b // chunk
    npad = n + 16
    mesh = plsc.VectorSubcoreMesh(core_axis_name="c", subcore_axis_name="s")
    init = jnp.full((npad * 8,), -jnp.inf, jnp.float32)
    msgflat = msg4.reshape(g_cnt * e * 8)
    idx8 = (idx[:, None] * 8
            + jnp.arange(8, dtype=jnp.int32)[None, :]).reshape(e * 8)

    @functools.partial(
        pl.kernel, mesh=mesh,
        compiler_params=pltpu.CompilerParams(needs_layout_passes=False),
        out_type=jax.ShapeDtypeStruct((_NW, npad * 8), jnp.float32),
        scratch_types=[
            pltpu.VMEM((chunk * 8,), jnp.int32),
            pltpu.VMEM((chunk * 8,), jnp.float32),
            pltpu.VMEM((npad * 8,), jnp.float32),
            pltpu.SemaphoreType.DMA,
        ],
    )
    def k(msg_hbm, idx_hbm, init_hbm, out_hbm, idx_v, val_v, acc_v, sem):
        wid = lax.axis_index("s") * 2 + lax.axis_index("c")
        g = wid // r_cnt
        r = wid % r_cnt
        pltpu.sync_copy(init_hbm, acc_v)
        lane = lax.iota(jnp.int32, 16)
        lo = lane < 8
        hi = ~lo

        def it(i, carry):
            fi = idx_v[pl.ds(i * 16, 16)]
            v = val_v[pl.ds(i * 16, 16)]
            cur = plsc.load_gather(acc_v, [fi])
            plsc.store_scatter(acc_v, [fi], jnp.maximum(cur, v), mask=lo)
            cur2 = plsc.load_gather(acc_v, [fi])
            plsc.store_scatter(acc_v, [fi], jnp.maximum(cur2, v), mask=hi)
            return carry

        for j in range(n_chunks):
            base = (r * b + j * chunk) * 8
            pltpu.sync_copy(idx_hbm.at[pl.ds(base, chunk * 8)], idx_v)
            pltpu.sync_copy(msg_hbm.at[pl.ds(g * (e * 8) + base, chunk * 8)],
                            val_v)
            lax.fori_loop(0, chunk // 2, it, 0)
        pltpu.sync_copy(acc_v, out_hbm.at[wid])

    out = k(msgflat, idx8, init)
    o = out.reshape(g_cnt, r_cnt, npad, 8)[:, :, :n, :]
    return jnp.moveaxis(o.max(axis=1), 0, 1).reshape(n, out_c)


def _spread_clamp(idx, n):
    fake = jnp.arange(idx.shape[0], dtype=jnp.int32) & 8191
    return jnp.where(idx < n, idx, fake % n)


def _msg_body(ea_ref, xs_ref, w1r_ref, b1r_ref, w2z_ref, out_ref, *, in_c):
    ea = ea_ref[...]
    e0 = ea[:, 0:1]
    e1 = ea[:, 1:2]
    hrep = jax.nn.relu(e0 * w1r_ref[0:1, :] + e1 * w1r_ref[1:2, :]
                       + b1r_ref[...])
    xs = xs_ref[...]
    xtile = jnp.concatenate([xs] * 26, axis=1)
    z = hrep * xtile
    msg = jnp.dot(z, w2z_ref[...], preferred_element_type=jnp.float32)
    for g in range(out_ref.shape[0]):
        out_ref[g] = msg[:, g * 8:(g + 1) * 8]


def _msg_call(ea, xs, W1, b1, W2, b2, in_c, out_c, block_e):
    e = ea.shape[0]
    k26 = 26 * in_c
    w2z = jnp.concatenate([W2, b2[None, :]], axis=0).reshape(26 * in_c, out_c)
    w2z = w2z.astype(jnp.float32)
    w1r = jnp.concatenate(
        [jnp.repeat(W1.astype(jnp.float32), in_c, axis=1),
         jnp.zeros((2, in_c), jnp.float32)], axis=1)
    b1r = jnp.concatenate(
        [jnp.repeat(b1.astype(jnp.float32), in_c),
         jnp.ones((in_c,), jnp.float32)])[None, :]
    body = functools.partial(_msg_body, in_c=in_c)
    return pl.pallas_call(
        body,
        grid=(e // block_e,),
        in_specs=[
            pl.BlockSpec((block_e, 2), lambda i: (i, 0)),
            pl.BlockSpec((block_e, in_c), lambda i: (i, 0)),
            pl.BlockSpec((2, k26), lambda i: (0, 0)),
            pl.BlockSpec((1, k26), lambda i: (0, 0)),
            pl.BlockSpec((k26, out_c), lambda i: (0, 0)),
        ],
        out_specs=pl.BlockSpec((out_c // 8, block_e, 8), lambda i: (0, i, 0)),
        out_shape=jax.ShapeDtypeStruct((out_c // 8, e, 8), jnp.float32),
    )(ea.astype(jnp.float32), xs, w1r, b1r, w2z)


def _norm_cut(dx, dy, deg_s, deg_d):
    w = jnp.sqrt(dx * dx + dy * dy + 1e-12)
    return w * (1.0 / deg_s + 1.0 / deg_d)


def _graclus_match(src, dst, w, n):
    row = jnp.concatenate([src, dst])
    col = jnp.concatenate([dst, src]).astype(jnp.int32)
    ww = jnp.concatenate([w, w])
    _, pr = _lexmax_scatter(row, col, ww, n)
    idx = jnp.arange(n, dtype=jnp.int32)
    partner = jnp.where(pr < 0, idx, pr)
    mutual = partner[partner] == idx
    return jnp.where(mutual, jnp.minimum(idx, partner), idx)


def _relabel_clusters(cluster, size):
    uniq, inv = jnp.unique(cluster, return_inverse=True, size=size,
                           fill_value=size)
    nc = (jnp.max(inv) + 1).astype(jnp.int32)
    return inv.astype(jnp.int32), nc, uniq


def _coalesce_edges(cl_s, cl_d, nc, n, e):
    m = cl_s != cl_d
    sent = n * n
    key = jnp.where(m, cl_s * nc + cl_d, sent)
    uniq = jnp.unique(key, size=e, fill_value=sent)
    valid = uniq < sent
    s2 = jnp.where(valid, uniq // nc, n).astype(jnp.int32)
    d2 = jnp.where(valid, uniq % nc, n).astype(jnp.int32)
    return s2, d2


def kernel(x, edge_index, edge_attr, pos, batch, nn1_W1, nn1_b1, nn1_W2,
           nn1_b2, root1, bias1, nn2_W1, nn2_b1, nn2_W2, nn2_b2, root2,
           bias2, fc1_W, fc1_b, fc2_W, fc2_b):
    n = x.shape[0]
    e = edge_index.shape[1]
    in_c = x.shape[1]
    h1 = root1.shape[1]
    h2 = root2.shape[1]
    src, dst = edge_index[0], edge_index[1]

    xs = x[src]
    msg4 = _msg_call(edge_attr, xs, nn1_W1, nn1_b1, nn1_W2, nn1_b2,
                     in_c, h1, 1280)
    agg = _segmax_rows(msg4, dst, n, h1)
    agg = jnp.where(jnp.isfinite(agg), agg, 0.0)
    x1 = jax.nn.elu(agg + x @ root1 + bias1)

    deg1 = jnp.clip(jnp.zeros((n,), jnp.float32).at[src].add(1.0), 1.0, None)
    px_s, py_s, px_d, py_d, deg_s, deg_d = _edge_geom(pos, deg1, src, dst, n)
    w1 = _norm_cut(px_s - px_d, py_s - py_d, deg_s, deg_d)
    cl1, nc1, _ = _relabel_clusters(_graclus_match(src, dst, w1, n), n)
    xp = jax.ops.segment_max(x1, cl1, num_segments=n)
    cnt1 = jnp.clip(jnp.zeros((n,), jnp.float32).at[cl1].add(1.0), 1.0, None)
    pos1 = jnp.zeros((n, 2), jnp.float32).at[cl1].add(pos) / cnt1[:, None]
    cl_s, cl_d = _gather_pair(cl1, src, dst)
    s2, d2 = _coalesce_edges(cl_s, cl_d, nc1, n, e)

    s2c = _spread_clamp(s2, n)
    d2c = _spread_clamp(d2, n)
    deg2 = jnp.clip(jnp.zeros((n,), jnp.float32).at[s2].add(1.0), 1.0, None)
    qx_s, qy_s, qx_d, qy_d, dg_s, dg_d = _edge_geom(pos1, deg2, s2c, d2c, n)
    cart = jnp.stack([qx_s - qx_d, qy_s - qy_d], axis=1)
    mx = jnp.max(jnp.abs(cart)) + 1e-12
    e2 = cart / (2.0 * mx) + 0.5
    xs2 = xp[jnp.minimum(s2, n - 1)]
    msg24 = _msg_call(e2, xs2, nn2_W1, nn2_b1, nn2_W2, nn2_b2, h1, h2, 1280)
    agg2 = _segmax_rows(msg24, d2, n, h2)
    agg2 = jnp.where(jnp.isfinite(agg2), agg2, 0.0)
    x2 = jax.nn.elu(agg2 + xp @ root2 + bias2)

    w2 = _norm_cut(qx_s - qx_d, qy_s - qy_d, dg_s, dg_d)
    cl2, _, uniq2 = _relabel_clusters(_graclus_match(s2, d2, w2, n), n)
    x3 = jax.ops.segment_max(x2, cl2, num_segments=n)
    valid2 = uniq2 < nc1
    g = jnp.sum(jnp.where(valid2[:, None], x3, 0.0), axis=0, keepdims=True)
    gc = jnp.clip(jnp.sum(valid2.astype(jnp.float32)), 1.0, None)
    g = g / gc

    h = jax.nn.elu(g @ fc1_W + fc1_b)
    out = h @ fc2_W + fc2_b
    return jax.nn.log_softmax(out, axis=1)


# speedup vs baseline: 1.3250x; 1.3250x over previous
import functools

import jax
import jax.numpy as jnp
from jax import lax
from jax.experimental import pallas as pl
from jax.experimental.pallas import tpu as pltpu
from jax.experimental.pallas import tpu_sc as plsc


_NW = 32


def _pick_chunk(b, d):
    best = None
    for div in (1, 2, 4, 5, 8, 10, 16, 20, 25, 40, 50):
        if b % div:
            continue
        c = b // div
        if c % 8 == 0 and c * (d + 1) * 4 <= 380_000:
            best = c
            break
    if best is None:
        raise ValueError(f"no chunk for b={b} d={d}")
    return best


def _sc_gather(table, idx):
    m = idx.shape[0]
    b = m // _NW
    two_d = table.ndim == 2
    d = table.shape[1] if two_d else 1
    c = _pick_chunk(b, d)
    n_chunks = b // c
    row_shape = (c, d) if two_d else (c,)
    mesh = plsc.VectorSubcoreMesh(core_axis_name="c", subcore_axis_name="s")

    @functools.partial(
        pl.kernel, mesh=mesh,
        out_type=jax.ShapeDtypeStruct(idx.shape[:1] + table.shape[1:],
                                      table.dtype),
        scratch_types=[
            pltpu.VMEM((c,), jnp.int32),
            pltpu.VMEM(row_shape, table.dtype),
            pltpu.SemaphoreType.DMA,
        ],
    )
    def k(table_hbm, idx_hbm, out_hbm, idx_v, rows_v, sem):
        wid = lax.axis_index("s") * 2 + lax.axis_index("c")
        base = wid * b
        for j in range(n_chunks):
            o = base + j * c
            pltpu.sync_copy(idx_hbm.at[pl.ds(o, c)], idx_v)
            pltpu.async_copy(table_hbm.at[idx_v], rows_v, sem).wait()
            pltpu.sync_copy(rows_v, out_hbm.at[pl.ds(o, c)])

    return k(table, idx)


def _gather_pair(table, ia, ib):
    out = _sc_gather(table, jnp.concatenate([ia, ib]))
    return out[: ia.shape[0]], out[ia.shape[0]:]


def _edge_geom(pos, deg, ia, ib, n):
    e = ia.shape[0]
    table = jnp.concatenate([pos[:, 0], pos[:, 1], deg.astype(jnp.float32)])
    idx = jnp.concatenate([ia, ib, n + ia, n + ib, 2 * n + ia, 2 * n + ib])
    out = _sc_gather(table, idx)
    px_s, px_d, py_s, py_d, dg_s, dg_d = [out[i * e:(i + 1) * e]
                                          for i in range(6)]
    return px_s, py_s, px_d, py_d, dg_s, dg_d


def _perm16(v, perm):
    dnums = lax.GatherDimensionNumbers(offset_dims=(),
                                       collapsed_slice_dims=(0,),
                                       start_index_map=(0,))
    return lax.gather(v, perm[:, None], dnums, slice_sizes=(1,),
                      mode=lax.GatherScatterMode.PROMISE_IN_BOUNDS)


def _lexmax_scatter(row, col, ww, n):
    m = row.shape[0]
    b = m // _NW
    npad = ((n + 256) // 256) * 256
    mesh = plsc.VectorSubcoreMesh(core_axis_name="c", subcore_axis_name="s")
    initw = jnp.full((npad,), -jnp.inf, jnp.float32)
    initc = jnp.full((npad,), -1, jnp.int32)

    @functools.partial(
        pl.kernel, mesh=mesh,
        compiler_params=pltpu.CompilerParams(needs_layout_passes=False),
        out_type=(jax.ShapeDtypeStruct((_NW, npad), jnp.float32),
                  jax.ShapeDtypeStruct((_NW, npad), jnp.int32)),
        scratch_types=[
            pltpu.VMEM((b,), jnp.int32),
            pltpu.VMEM((b,), jnp.int32),
            pltpu.VMEM((b,), jnp.float32),
            pltpu.VMEM((npad,), jnp.float32),
            pltpu.VMEM((npad,), jnp.int32),
            pltpu.SemaphoreType.DMA,
        ],
    )
    def k(row_hbm, col_hbm, ww_hbm, iw_hbm, ic_hbm, outw_hbm, outc_hbm,
          idx_v, col_v, ww_v, accw_v, accc_v, sem):
        wid = lax.axis_index("s") * 2 + lax.axis_index("c")
        base = wid * b
        pltpu.sync_copy(iw_hbm, accw_v)
        pltpu.sync_copy(ic_hbm, accc_v)
        pltpu.sync_copy(row_hbm.at[pl.ds(base, b)], idx_v)
        pltpu.sync_copy(col_hbm.at[pl.ds(base, b)], col_v)
        pltpu.sync_copy(ww_hbm.at[pl.ds(base, b)], ww_v)
        lane = lax.iota(jnp.int32, 16)

        def it(i, carry):
            s = pl.ds(i * 16, 16)
            key = idx_v[s] * 16384 + col_v[s]
            ks, ws = plsc.sort_key_val(key, ww_v[s])
            seg = ks >> 14
            cs = ks & 16383
            for d in (1, 2, 4, 8):
                perm = jnp.maximum(lane - d, 0)
                pi = _perm16(seg, perm)
                pw = _perm16(ws, perm)
                pc = _perm16(cs, perm)
                same = pi == seg
                win = same & ((pw > ws) | ((pw == ws) & (pc > cs)))
                ws = jnp.where(win, pw, ws)
                cs = jnp.where(win, pc, cs)
            nxt = _perm16(seg, jnp.minimum(lane + 1, 15))
            last = (seg != nxt) | (lane == 15)
            cw = plsc.load_gather(accw_v, [seg])
            cc = plsc.load_gather(accc_v, [seg])
            win2 = (ws > cw) | ((ws == cw) & (cs > cc))
            msk = last & win2
            plsc.store_scatter(accw_v, [seg], ws, mask=msk)
            plsc.store_scatter(accc_v, [seg], cs, mask=msk)
            return carry

        lax.fori_loop(0, b // 16, it, 0)
        pltpu.sync_copy(accw_v, outw_hbm.at[wid])
        pltpu.sync_copy(accc_v, outc_hbm.at[wid])

    outw, outc = k(row, col, ww, initw, initc)
    mw = jnp.max(outw[:, :n], axis=0)
    pr = jnp.max(jnp.where(outw[:, :n] == mw[None, :], outc[:, :n], -1),
                 axis=0)
    return mw, pr


def _spread_clamp(idx, n):
    fake = jnp.arange(idx.shape[0], dtype=jnp.int32) & 8191
    return jnp.where(idx < n, idx, fake % n)


def _msg_body(ea_ref, xs_ref, w1r_ref, b1r_ref, w2z_ref, out_ref, *, in_c):
    ea = ea_ref[...]
    e0 = ea[:, 0:1]
    e1 = ea[:, 1:2]
    hrep = jax.nn.relu(e0 * w1r_ref[0:1, :] + e1 * w1r_ref[1:2, :]
                       + b1r_ref[...])
    xs = xs_ref[...]
    xtile = jnp.concatenate([xs] * 26, axis=1)
    z = hrep * xtile
    out_ref[...] = jnp.dot(z, w2z_ref[...],
                           preferred_element_type=jnp.float32)


def _msg_call(ea, xs, W1, b1, W2, b2, in_c, out_c, block_e):
    e = ea.shape[0]
    k26 = 26 * in_c
    w2z = jnp.concatenate([W2, b2[None, :]], axis=0).reshape(26 * in_c, out_c)
    w2z = w2z.astype(jnp.float32)
    w1r = jnp.concatenate(
        [jnp.repeat(W1.astype(jnp.float32), in_c, axis=1),
         jnp.zeros((2, in_c), jnp.float32)], axis=1)
    b1r = jnp.concatenate(
        [jnp.repeat(b1.astype(jnp.float32), in_c),
         jnp.ones((in_c,), jnp.float32)])[None, :]
    body = functools.partial(_msg_body, in_c=in_c)
    return pl.pallas_call(
        body,
        grid=(e // block_e,),
        in_specs=[
            pl.BlockSpec((block_e, 2), lambda i: (i, 0)),
            pl.BlockSpec((block_e, in_c), lambda i: (i, 0)),
            pl.BlockSpec((2, k26), lambda i: (0, 0)),
            pl.BlockSpec((1, k26), lambda i: (0, 0)),
            pl.BlockSpec((k26, out_c), lambda i: (0, 0)),
        ],
        out_specs=pl.BlockSpec((block_e, out_c), lambda i: (i, 0)),
        out_shape=jax.ShapeDtypeStruct((e, out_c), jnp.float32),
    )(ea.astype(jnp.float32), xs, w1r, b1r, w2z)


def _norm_cut(dx, dy, deg_s, deg_d):
    w = jnp.sqrt(dx * dx + dy * dy + 1e-12)
    return w * (1.0 / deg_s + 1.0 / deg_d)


def _graclus_match(src, dst, w, n):
    row = jnp.concatenate([src, dst])
    col = jnp.concatenate([dst, src]).astype(jnp.int32)
    ww = jnp.concatenate([w, w])
    _, pr = _lexmax_scatter(row, col, ww, n)
    idx = jnp.arange(n, dtype=jnp.int32)
    partner = jnp.where(pr < 0, idx, pr)
    mutual = partner[partner] == idx
    return jnp.where(mutual, jnp.minimum(idx, partner), idx)


def _relabel_clusters(cluster, size):
    uniq, inv = jnp.unique(cluster, return_inverse=True, size=size,
                           fill_value=size)
    nc = (jnp.max(inv) + 1).astype(jnp.int32)
    return inv.astype(jnp.int32), nc, uniq


def _coalesce_edges(cl_s, cl_d, nc, n, e):
    m = cl_s != cl_d
    sent = n * n
    key = jnp.where(m, cl_s * nc + cl_d, sent)
    uniq = jnp.unique(key, size=e, fill_value=sent)
    valid = uniq < sent
    s2 = jnp.where(valid, uniq // nc, n).astype(jnp.int32)
    d2 = jnp.where(valid, uniq % nc, n).astype(jnp.int32)
    return s2, d2


def kernel(x, edge_index, edge_attr, pos, batch, nn1_W1, nn1_b1, nn1_W2,
           nn1_b2, root1, bias1, nn2_W1, nn2_b1, nn2_W2, nn2_b2, root2,
           bias2, fc1_W, fc1_b, fc2_W, fc2_b):
    n = x.shape[0]
    e = edge_index.shape[1]
    in_c = x.shape[1]
    h1 = root1.shape[1]
    h2 = root2.shape[1]
    src, dst = edge_index[0], edge_index[1]

    xs = x[src]
    msg = _msg_call(edge_attr, xs, nn1_W1, nn1_b1, nn1_W2, nn1_b2,
                    in_c, h1, 1280)
    agg = jax.ops.segment_max(msg, dst, num_segments=n)
    agg = jnp.where(jnp.isfinite(agg), agg, 0.0)
    x1 = jax.nn.elu(agg + x @ root1 + bias1)

    deg1 = jnp.clip(jnp.zeros((n,), jnp.float32).at[src].add(1.0), 1.0, None)
    px_s, py_s, px_d, py_d, deg_s, deg_d = _edge_geom(pos, deg1, src, dst, n)
    w1 = _norm_cut(px_s - px_d, py_s - py_d, deg_s, deg_d)
    cl1, nc1, _ = _relabel_clusters(_graclus_match(src, dst, w1, n), n)
    xp = jax.ops.segment_max(x1, cl1, num_segments=n)
    cnt1 = jnp.clip(jnp.zeros((n,), jnp.float32).at[cl1].add(1.0), 1.0, None)
    pos1 = jnp.zeros((n, 2), jnp.float32).at[cl1].add(pos) / cnt1[:, None]
    cl_s, cl_d = _gather_pair(cl1, src, dst)
    s2, d2 = _coalesce_edges(cl_s, cl_d, nc1, n, e)

    s2c = _spread_clamp(s2, n)
    d2c = _spread_clamp(d2, n)
    deg2 = jnp.clip(jnp.zeros((n,), jnp.float32).at[s2].add(1.0), 1.0, None)
    qx_s, qy_s, qx_d, qy_d, dg_s, dg_d = _edge_geom(pos1, deg2, s2c, d2c, n)
    cart = jnp.stack([qx_s - qx_d, qy_s - qy_d], axis=1)
    mx = jnp.max(jnp.abs(cart)) + 1e-12
    e2 = cart / (2.0 * mx) + 0.5
    xs2 = xp[jnp.minimum(s2, n - 1)]
    msg2 = _msg_call(e2, xs2, nn2_W1, nn2_b1, nn2_W2, nn2_b2, h1, h2, 1280)
    agg2 = jax.ops.segment_max(msg2, d2, num_segments=n)
    agg2 = jnp.where(jnp.isfinite(agg2), agg2, 0.0)
    x2 = jax.nn.elu(agg2 + xp @ root2 + bias2)

    w2 = _norm_cut(qx_s - qx_d, qy_s - qy_d, dg_s, dg_d)
    cl2, _, uniq2 = _relabel_clusters(_graclus_match(s2, d2, w2, n), n)
    x3 = jax.ops.segment_max(x2, cl2, num_segments=n)
    valid2 = uniq2 < nc1
    g = jnp.sum(jnp.where(valid2[:, None], x3, 0.0), axis=0, keepdims=True)
    gc = jnp.clip(jnp.sum(valid2.astype(jnp.float32)), 1.0, None)
    g = g / gc

    h = jax.nn.elu(g @ fc1_W + fc1_b)
    out = h @ fc2_W + fc2_b
    return jax.nn.log_softmax(out, axis=1)
